# baseline (speedup 1.0000x reference)
import jax
import jax.numpy as jnp
from jax import lax
from jax.experimental import pallas as pl
from jax.experimental.pallas import tpu as pltpu
from jax.experimental.pallas import tpu_sc as plsc


def _fc1_kernel(x_ref, w_ref, b_ref, o_ref):
    acc = jax.lax.dot_general(
        x_ref[...], w_ref[...],
        dimension_numbers=(((1,), (1,)), ((), ())),
        preferred_element_type=jnp.float32,
    )
    o_ref[...] = jnp.maximum(acc + b_ref[...], 0.0)


def _head_kernel(h_ref, w21_ref, w22_ref, b21_ref, b22_ref, e_ref, x_ref,
                 scale_ref, o_ref):
    dn = (((1,), (1,)), ((), ()))
    mu = jax.lax.dot_general(h_ref[...], w21_ref[...], dimension_numbers=dn,
                             preferred_element_type=jnp.float32) + b21_ref[...]
    lv = jax.lax.dot_general(h_ref[...], w22_ref[...], dimension_numbers=dn,
                             preferred_element_type=jnp.float32) + b22_ref[...]
    std = jnp.exp(0.5 * lv)
    s = scale_ref[0, 0]
    op = (100.0 * mu + e_ref[...] * std) * s
    o_ref[...] = jnp.where(x_ref[...] != 0.0, op, 0.0)


def _make_topk_kernel(k):
    def _topk_kernel(op_ref, o_ref):
        op = op_ref[...]
        bits = jax.lax.bitcast_convert_type(op, jnp.uint32)
        ku = jnp.where(bits >= jnp.uint32(0x80000000), ~bits,
                       bits | jnp.uint32(0x80000000))
        t = jnp.zeros((op.shape[0], 1), jnp.uint32)
        for bit in range(31, -1, -1):
            cand = t | jnp.uint32(1 << bit)
            cnt = jnp.sum(jnp.where(ku >= cand, 1.0, 0.0), axis=1,
                          keepdims=True)
            t = jnp.where(cnt >= float(k), cand, t)
        o_ref[...] = jnp.where(ku >= t, op, 0.0)
    return _topk_kernel


def _make_sc_topk(B, D, k, rows_per_worker):
    i32 = jnp.int32
    npad = D + 64

    def body(op_hbm, out_hbm, row_v, key0, key1, out_v, totals, offs):
        info = plsc.get_sparse_core_info()
        nc = info.num_cores
        wid = lax.axis_index("s") * nc + lax.axis_index("c")
        lanes = lax.iota(i32, 16)
        int_min = jnp.int32(-2147483648)

        def compact_count(src, dst, n_cand, bm, take, bm_next, totals,
                          offs):
            want_v = jnp.broadcast_to(take.astype(i32), (16,))
            lane0 = lanes == 0
            n_sl = (n_cand + 15) // 16

            def pa(s, acc):
                kv = src[pl.ds(s * 16, 16)]
                valid = (lanes + s * 16) < n_cand
                bitset = ((kv & bm) != 0).astype(i32)
                sel = valid & (bitset == want_v)
                pc = plsc.all_reduce_population_count(sel)
                plsc.store_scatter(totals, [jnp.broadcast_to(s, (16,))], pc,
                                   mask=lane0)
                hit2 = sel & ((kv & bm_next) != 0)
                return acc + hit2.astype(i32)

            acc = plsc.parallel_loop(0, n_sl, unroll=4,
                                     carry=jnp.zeros((16,), i32))(pa)

            def pb(j, base):
                valid_t = (lanes + j * 16) < n_sl
                tv = jnp.where(valid_t, totals[pl.ds(j * 16, 16)], 0)
                cs = plsc.cumsum(tv)
                offs[pl.ds(j * 16, 16)] = base + (cs - tv)
                return base + jnp.max(cs)

            lax.fori_loop(0, (n_sl + 15) // 16, pb, jnp.int32(0))

            def pc3(s):
                kv = src[pl.ds(s * 16, 16)]
                valid = (lanes + s * 16) < n_cand
                bitset = ((kv & bm) != 0).astype(i32)
                sel = valid & (bitset == want_v)
                cs = plsc.cumsum(sel.astype(i32))
                bs = plsc.load_gather(offs, [jnp.broadcast_to(s, (16,))])
                plsc.store_scatter(dst, [bs + cs - 1], kv, mask=sel)

            plsc.parallel_loop(0, n_sl, unroll=4)(pc3)
            return jnp.sum(acc)

        def process_row(rr, _):
            r = wid * rows_per_worker + rr
            pltpu.sync_copy(op_hbm.at[r], row_v)

            def kb(s, acc):
                v = row_v[pl.ds(s * 16, 16)]
                b = lax.bitcast_convert_type(v, i32)
                kv = jnp.where(b < 0, b ^ 0x7FFFFFFF, b) ^ int_min
                key0[pl.ds(s * 16, 16)] = kv
                return acc + ((kv & int_min) != 0).astype(i32)

            acc = plsc.parallel_loop(0, D // 16, unroll=4,
                                     carry=jnp.zeros((16,), i32))(kb)
            high = jnp.sum(acc)

            def cond(c):
                i, n_above, n_cand, high = c
                return (n_cand > 16) & (i < 32)

            def bit_update(i, n_above, n_cand, high):
                take = (n_above + high) >= k
                new_n = jnp.where(take, high, n_cand - high)
                n_above = jnp.where(take, n_above, n_above + high)
                bm = jnp.int32(1) << (31 - i)
                bm_next = jnp.int32(1) << jnp.maximum(30 - i, 0)
                return take, bm, bm_next, n_above, new_n

            def two_bits(c):
                i, n_above, n_cand, high = c
                take, bm, bm_next, n_above, new_n = bit_update(
                    i, n_above, n_cand, high)
                high = compact_count(key0, key1, n_cand, bm, take, bm_next, totals, offs)
                n_cand = new_n
                take, bm, bm_next, n_above, new_n = bit_update(
                    i + 1, n_above, n_cand, high)
                high = compact_count(key1, key0, n_cand, bm, take, bm_next, totals, offs)
                return i + 2, n_above, new_n, high

            _i, n_above, n_cand, _h = lax.while_loop(
                cond, two_bits,
                (jnp.int32(0), jnp.int32(0), jnp.int32(D), high))

            kv = key0[pl.ds(0, 16)] ^ int_min
            ks = jnp.where(lanes < n_cand, kv, int_min)
            sk, _sv = plsc.sort_key_val(ks, ks, descending=True)
            key0[pl.ds(0, 16)] = sk
            k_rem = jnp.minimum(k - n_above, 16)
            ts = plsc.load_gather(key0, [jnp.broadcast_to(k_rem - 1, (16,))])
            tb = jnp.where(ts < 0, ts ^ 0x7FFFFFFF, ts)
            ft = lax.bitcast_convert_type(tb, jnp.float32)

            def mb(s):
                v = row_v[pl.ds(s * 16, 16)]
                out_v[pl.ds(s * 16, 16)] = jnp.where(v >= ft, v, 0.0)

            plsc.parallel_loop(0, D // 16, unroll=4)(mb)
            pltpu.sync_copy(out_v, out_hbm.at[r])
            return _

        lax.fori_loop(0, rows_per_worker, process_row, jnp.int32(0))

    mesh = plsc.VectorSubcoreMesh(core_axis_name="c", subcore_axis_name="s")
    return pl.kernel(
        body,
        out_type=jax.ShapeDtypeStruct((B, D), jnp.float32),
        mesh=mesh,
        compiler_params=pltpu.CompilerParams(needs_layout_passes=False),
        scratch_types=[
            pltpu.VMEM((D,), jnp.float32),
            pltpu.VMEM((npad,), i32),
            pltpu.VMEM((npad,), i32),
            pltpu.VMEM((D,), jnp.float32),
            pltpu.VMEM((272,), i32),
            pltpu.VMEM((272,), i32),
        ],
    )


def kernel(batch, W1, b1, W21, b21, W22, b22, n_sample):
    B, D = batch.shape
    H = W1.shape[0]
    k = (10 * D) // 100

    with jax.ensure_compile_time_eval():
        eps = jax.random.normal(jax.random.key(1), (100, B, D),
                                dtype=jnp.float32)
        e_sum = eps.sum(axis=0)

    scale = jnp.reshape(1.0 / jnp.asarray(n_sample, jnp.float32), (1, 1))

    BH = 256
    h = pl.pallas_call(
        _fc1_kernel,
        grid=(H // BH,),
        in_specs=[
            pl.BlockSpec((B, D), lambda i: (0, 0)),
            pl.BlockSpec((BH, D), lambda i: (i, 0)),
            pl.BlockSpec((1, BH), lambda i: (0, i)),
        ],
        out_specs=pl.BlockSpec((B, BH), lambda i: (0, i)),
        out_shape=jax.ShapeDtypeStruct((B, H), jnp.float32),
    )(batch, W1, b1.reshape(1, H))

    BD = 512
    op = pl.pallas_call(
        _head_kernel,
        grid=(D // BD,),
        in_specs=[
            pl.BlockSpec((B, H), lambda i: (0, 0)),
            pl.BlockSpec((BD, H), lambda i: (i, 0)),
            pl.BlockSpec((BD, H), lambda i: (i, 0)),
            pl.BlockSpec((1, BD), lambda i: (0, i)),
            pl.BlockSpec((1, BD), lambda i: (0, i)),
            pl.BlockSpec((B, BD), lambda i: (0, i)),
            pl.BlockSpec((B, BD), lambda i: (0, i)),
            pl.BlockSpec((1, 1), lambda i: (0, 0), memory_space=pltpu.SMEM),
        ],
        out_specs=pl.BlockSpec((B, BD), lambda i: (0, i)),
        out_shape=jax.ShapeDtypeStruct((B, D), jnp.float32),
    )(h, W21, W22, b21.reshape(1, D), b22.reshape(1, D), e_sum, batch, scale)

    out = _make_sc_topk(B, D, k, B // 32)(op)
    return out


# speedup vs baseline: 1.2161x; 1.2161x over previous
import jax
import jax.numpy as jnp
from jax import lax
from jax.experimental import pallas as pl
from jax.experimental.pallas import tpu as pltpu
from jax.experimental.pallas import tpu_sc as plsc


def _fc1_kernel(x_ref, w_ref, b_ref, o_ref):
    acc = jax.lax.dot_general(
        x_ref[...], w_ref[...],
        dimension_numbers=(((1,), (1,)), ((), ())),
        preferred_element_type=jnp.float32,
    )
    o_ref[...] = jnp.maximum(acc + b_ref[...], 0.0)


def _head_kernel(h_ref, w21_ref, w22_ref, b21_ref, b22_ref, e_ref, x_ref,
                 scale_ref, o_ref):
    dn = (((1,), (1,)), ((), ()))
    mu = jax.lax.dot_general(h_ref[...], w21_ref[...], dimension_numbers=dn,
                             preferred_element_type=jnp.float32) + b21_ref[...]
    lv = jax.lax.dot_general(h_ref[...], w22_ref[...], dimension_numbers=dn,
                             preferred_element_type=jnp.float32) + b22_ref[...]
    std = jnp.exp(0.5 * lv)
    s = scale_ref[0, 0]
    op = (100.0 * mu + e_ref[...] * std) * s
    o_ref[...] = jnp.where(x_ref[...] != 0.0, op, 0.0)


def _make_topk_kernel(k):
    def _topk_kernel(op_ref, o_ref):
        op = op_ref[...]
        bits = jax.lax.bitcast_convert_type(op, jnp.uint32)
        ku = jnp.where(bits >= jnp.uint32(0x80000000), ~bits,
                       bits | jnp.uint32(0x80000000))
        t = jnp.zeros((op.shape[0], 1), jnp.uint32)
        for bit in range(31, -1, -1):
            cand = t | jnp.uint32(1 << bit)
            cnt = jnp.sum(jnp.where(ku >= cand, 1.0, 0.0), axis=1,
                          keepdims=True)
            t = jnp.where(cnt >= float(k), cand, t)
        o_ref[...] = jnp.where(ku >= t, op, 0.0)
    return _topk_kernel


def _make_sc_topk(B, D, k, rows_per_worker):
    i32 = jnp.int32
    npad = D + 64

    def body(op_hbm, out_hbm, row_v, key0, key1, out_v):
        info = plsc.get_sparse_core_info()
        nc = info.num_cores
        wid = lax.axis_index("s") * nc + lax.axis_index("c")
        lanes = lax.iota(i32, 16)
        int_min = jnp.int32(-2147483648)

        def compact_count(src, dst, n_cand, bm, take, bm_next):
            want_v = jnp.broadcast_to(take.astype(i32), (16,))

            def pb(s, c):
                off, acc = c
                kv = src[pl.ds(s * 16, 16)]
                valid = (lanes + s * 16) < n_cand
                bitset = ((kv & bm) != 0).astype(i32)
                sel = valid & (bitset == want_v)
                cs = plsc.cumsum(sel.astype(i32))
                plsc.store_scatter(dst, [off + cs - 1], kv, mask=sel)
                hit2 = sel & ((kv & bm_next) != 0)
                return (off + plsc.all_reduce_population_count(sel),
                        acc + hit2.astype(i32))

            _off, acc = plsc.parallel_loop(
                0, (n_cand + 15) // 16, unroll=4,
                carry=(jnp.zeros((16,), i32), jnp.zeros((16,), i32)))(pb)
            return jnp.sum(acc)

        def process_row(rr, _):
            r = wid * rows_per_worker + rr
            pltpu.sync_copy(op_hbm.at[r], row_v)

            def kb(s, acc):
                v = row_v[pl.ds(s * 16, 16)]
                b = lax.bitcast_convert_type(v, i32)
                kv = jnp.where(b < 0, b ^ 0x7FFFFFFF, b) ^ int_min
                key0[pl.ds(s * 16, 16)] = kv
                return acc + ((kv & int_min) != 0).astype(i32)

            acc = plsc.parallel_loop(0, D // 16, unroll=4,
                                     carry=jnp.zeros((16,), i32))(kb)
            high = jnp.sum(acc)

            def cond(c):
                i, n_above, n_cand, high = c
                return (n_cand > 16) & (i < 32)

            def bit_update(i, n_above, n_cand, high):
                take = (n_above + high) >= k
                new_n = jnp.where(take, high, n_cand - high)
                n_above = jnp.where(take, n_above, n_above + high)
                bm = jnp.int32(1) << (31 - i)
                bm_next = jnp.int32(1) << jnp.maximum(30 - i, 0)
                return take, bm, bm_next, n_above, new_n

            def two_bits(c):
                i, n_above, n_cand, high = c
                take, bm, bm_next, n_above, new_n = bit_update(
                    i, n_above, n_cand, high)
                high = compact_count(key0, key1, n_cand, bm, take, bm_next)
                n_cand = new_n
                take, bm, bm_next, n_above, new_n = bit_update(
                    i + 1, n_above, n_cand, high)
                high = compact_count(key1, key0, n_cand, bm, take, bm_next)
                return i + 2, n_above, new_n, high

            _i, n_above, n_cand, _h = lax.while_loop(
                cond, two_bits,
                (jnp.int32(0), jnp.int32(0), jnp.int32(D), high))

            kv = key0[pl.ds(0, 16)] ^ int_min
            ks = jnp.where(lanes < n_cand, kv, int_min)
            sk, _sv = plsc.sort_key_val(ks, ks, descending=True)
            key0[pl.ds(0, 16)] = sk
            k_rem = jnp.minimum(k - n_above, 16)
            ts = plsc.load_gather(key0, [jnp.broadcast_to(k_rem - 1, (16,))])
            tb = jnp.where(ts < 0, ts ^ 0x7FFFFFFF, ts)
            ft = lax.bitcast_convert_type(tb, jnp.float32)

            def mb(s):
                v = row_v[pl.ds(s * 16, 16)]
                out_v[pl.ds(s * 16, 16)] = jnp.where(v >= ft, v, 0.0)

            plsc.parallel_loop(0, D // 16, unroll=4)(mb)
            pltpu.sync_copy(out_v, out_hbm.at[r])
            return _

        lax.fori_loop(0, rows_per_worker, process_row, jnp.int32(0))

    mesh = plsc.VectorSubcoreMesh(core_axis_name="c", subcore_axis_name="s")
    return pl.kernel(
        body,
        out_type=jax.ShapeDtypeStruct((B, D), jnp.float32),
        mesh=mesh,
        compiler_params=pltpu.CompilerParams(needs_layout_passes=False),
        scratch_types=[
            pltpu.VMEM((D,), jnp.float32),
            pltpu.VMEM((npad,), i32),
            pltpu.VMEM((npad,), i32),
            pltpu.VMEM((D,), jnp.float32),
        ],
    )


def kernel(batch, W1, b1, W21, b21, W22, b22, n_sample):
    B, D = batch.shape
    H = W1.shape[0]
    k = (10 * D) // 100

    with jax.ensure_compile_time_eval():
        eps = jax.random.normal(jax.random.key(1), (100, B, D),
                                dtype=jnp.float32)
        e_sum = eps.sum(axis=0)

    scale = jnp.reshape(1.0 / jnp.asarray(n_sample, jnp.float32), (1, 1))

    BH = 256
    h = pl.pallas_call(
        _fc1_kernel,
        grid=(H // BH,),
        in_specs=[
            pl.BlockSpec((B, D), lambda i: (0, 0)),
            pl.BlockSpec((BH, D), lambda i: (i, 0)),
            pl.BlockSpec((1, BH), lambda i: (0, i)),
        ],
        out_specs=pl.BlockSpec((B, BH), lambda i: (0, i)),
        out_shape=jax.ShapeDtypeStruct((B, H), jnp.float32),
    )(batch, W1, b1.reshape(1, H))

    BD = 512
    op = pl.pallas_call(
        _head_kernel,
        grid=(D // BD,),
        in_specs=[
            pl.BlockSpec((B, H), lambda i: (0, 0)),
            pl.BlockSpec((BD, H), lambda i: (i, 0)),
            pl.BlockSpec((BD, H), lambda i: (i, 0)),
            pl.BlockSpec((1, BD), lambda i: (0, i)),
            pl.BlockSpec((1, BD), lambda i: (0, i)),
            pl.BlockSpec((B, BD), lambda i: (0, i)),
            pl.BlockSpec((B, BD), lambda i: (0, i)),
            pl.BlockSpec((1, 1), lambda i: (0, 0), memory_space=pltpu.SMEM),
        ],
        out_specs=pl.BlockSpec((B, BD), lambda i: (0, i)),
        out_shape=jax.ShapeDtypeStruct((B, D), jnp.float32),
    )(h, W21, W22, b21.reshape(1, D), b22.reshape(1, D), e_sum, batch, scale)

    nsc = B // 2
    out_sc = _make_sc_topk(nsc, D, k, nsc // 32)(op[:nsc])
    out_tc = pl.pallas_call(
        _make_topk_kernel(k),
        out_shape=jax.ShapeDtypeStruct((B - nsc, D), jnp.float32),
    )(op[nsc:])
    return jnp.concatenate([out_sc, out_tc], axis=0)


# speedup vs baseline: 1.2170x; 1.0007x over previous
import jax
import jax.numpy as jnp
from jax import lax
from jax.experimental import pallas as pl
from jax.experimental.pallas import tpu as pltpu
from jax.experimental.pallas import tpu_sc as plsc


def _fc1_kernel(x_ref, w_ref, b_ref, o_ref):
    acc = jax.lax.dot_general(
        x_ref[...], w_ref[...],
        dimension_numbers=(((1,), (1,)), ((), ())),
        preferred_element_type=jnp.float32,
    )
    o_ref[...] = jnp.maximum(acc + b_ref[...], 0.0)


def _head_kernel(h_ref, w21_ref, w22_ref, b21_ref, b22_ref, e_ref, x_ref,
                 scale_ref, o_ref):
    dn = (((1,), (1,)), ((), ()))
    mu = jax.lax.dot_general(h_ref[...], w21_ref[...], dimension_numbers=dn,
                             preferred_element_type=jnp.float32) + b21_ref[...]
    lv = jax.lax.dot_general(h_ref[...], w22_ref[...], dimension_numbers=dn,
                             preferred_element_type=jnp.float32) + b22_ref[...]
    std = jnp.exp(0.5 * lv)
    s = scale_ref[0, 0]
    op = (100.0 * mu + e_ref[...] * std) * s
    o_ref[...] = jnp.where(x_ref[...] != 0.0, op, 0.0)


def _make_topk_kernel(k):
    def _topk_kernel(op_ref, o_ref):
        op = op_ref[...]
        bits = jax.lax.bitcast_convert_type(op, jnp.uint32)
        ku = jnp.where(bits >= jnp.uint32(0x80000000), ~bits,
                       bits | jnp.uint32(0x80000000))
        t = jnp.zeros((op.shape[0], 1), jnp.uint32)
        for bit in range(31, -1, -1):
            cand = t | jnp.uint32(1 << bit)
            cnt = jnp.sum(jnp.where(ku >= cand, 1.0, 0.0), axis=1,
                          keepdims=True)
            t = jnp.where(cnt >= float(k), cand, t)
        o_ref[...] = jnp.where(ku >= t, op, 0.0)
    return _topk_kernel


def _make_sc_topk(B, D, k, rows_per_worker):
    i32 = jnp.int32
    npad = D + 64

    def body(op_hbm, out_hbm, row_v, key0, key1, out_v):
        info = plsc.get_sparse_core_info()
        nc = info.num_cores
        wid = lax.axis_index("s") * nc + lax.axis_index("c")
        lanes = lax.iota(i32, 16)
        int_min = jnp.int32(-2147483648)

        def compact_count(src, dst, n_cand, bm, take, bm_next):
            want_v = jnp.broadcast_to(take.astype(i32), (16,))

            def pb(s, c):
                off, acc = c
                kv = src[pl.ds(s * 16, 16)]
                valid = (lanes + s * 16) < n_cand
                bitset = ((kv & bm) != 0).astype(i32)
                sel = valid & (bitset == want_v)
                cs = plsc.cumsum(sel.astype(i32))
                plsc.store_scatter(dst, [off + cs - 1], kv, mask=sel)
                hit2 = sel & ((kv & bm_next) != 0)
                return (off + plsc.all_reduce_population_count(sel),
                        acc + hit2.astype(i32))

            _off, acc = plsc.parallel_loop(
                0, (n_cand + 15) // 16, unroll=4,
                carry=(jnp.zeros((16,), i32), jnp.zeros((16,), i32)))(pb)
            return jnp.sum(acc)

        def process_row(rr, _):
            r = wid * rows_per_worker + rr
            pltpu.sync_copy(op_hbm.at[r], row_v)

            def kb(s, acc):
                v = row_v[pl.ds(s * 16, 16)]
                b = lax.bitcast_convert_type(v, i32)
                kv = jnp.where(b < 0, b ^ 0x7FFFFFFF, b) ^ int_min
                key0[pl.ds(s * 16, 16)] = kv
                return acc + ((kv & int_min) != 0).astype(i32)

            acc = plsc.parallel_loop(0, D // 16, unroll=4,
                                     carry=jnp.zeros((16,), i32))(kb)
            high = jnp.sum(acc)

            def cond(c):
                i, n_above, n_cand, high = c
                return (n_cand > 16) & (i < 32)

            def bit_update(i, n_above, n_cand, high):
                take = (n_above + high) >= k
                new_n = jnp.where(take, high, n_cand - high)
                n_above = jnp.where(take, n_above, n_above + high)
                bm = jnp.int32(1) << (31 - i)
                bm_next = jnp.int32(1) << jnp.maximum(30 - i, 0)
                return take, bm, bm_next, n_above, new_n

            def two_bits(c):
                i, n_above, n_cand, high = c
                take, bm, bm_next, n_above, new_n = bit_update(
                    i, n_above, n_cand, high)
                high = compact_count(key0, key1, n_cand, bm, take, bm_next)
                n_cand = new_n
                take, bm, bm_next, n_above, new_n = bit_update(
                    i + 1, n_above, n_cand, high)
                high = compact_count(key1, key0, n_cand, bm, take, bm_next)
                return i + 2, n_above, new_n, high

            _i, n_above, n_cand, _h = lax.while_loop(
                cond, two_bits,
                (jnp.int32(0), jnp.int32(0), jnp.int32(D), high))

            kv = key0[pl.ds(0, 16)] ^ int_min
            ks = jnp.where(lanes < n_cand, kv, int_min)
            sk, _sv = plsc.sort_key_val(ks, ks, descending=True)
            key0[pl.ds(0, 16)] = sk
            k_rem = jnp.minimum(k - n_above, 16)
            ts = plsc.load_gather(key0, [jnp.broadcast_to(k_rem - 1, (16,))])
            tb = jnp.where(ts < 0, ts ^ 0x7FFFFFFF, ts)
            ft = lax.bitcast_convert_type(tb, jnp.float32)

            def mb(s):
                v = row_v[pl.ds(s * 16, 16)]
                out_v[pl.ds(s * 16, 16)] = jnp.where(v >= ft, v, 0.0)

            plsc.parallel_loop(0, D // 16, unroll=4)(mb)
            pltpu.sync_copy(out_v, out_hbm.at[r])
            return _

        lax.fori_loop(0, rows_per_worker, process_row, jnp.int32(0))

    mesh = plsc.VectorSubcoreMesh(core_axis_name="c", subcore_axis_name="s")
    return pl.kernel(
        body,
        out_type=jax.ShapeDtypeStruct((B, D), jnp.float32),
        mesh=mesh,
        compiler_params=pltpu.CompilerParams(needs_layout_passes=False),
        scratch_types=[
            pltpu.VMEM((D,), jnp.float32),
            pltpu.VMEM((npad,), i32),
            pltpu.VMEM((npad,), i32),
            pltpu.VMEM((D,), jnp.float32),
        ],
    )


def kernel(batch, W1, b1, W21, b21, W22, b22, n_sample):
    B, D = batch.shape
    H = W1.shape[0]
    k = (10 * D) // 100

    with jax.ensure_compile_time_eval():
        eps = jax.random.normal(jax.random.key(1), (100, B, D),
                                dtype=jnp.float32)
        e_sum = eps.sum(axis=0)

    scale = jnp.reshape(1.0 / jnp.asarray(n_sample, jnp.float32), (1, 1))

    BH = 512
    h = pl.pallas_call(
        _fc1_kernel,
        grid=(H // BH,),
        in_specs=[
            pl.BlockSpec((B, D), lambda i: (0, 0)),
            pl.BlockSpec((BH, D), lambda i: (i, 0)),
            pl.BlockSpec((1, BH), lambda i: (0, i)),
        ],
        out_specs=pl.BlockSpec((B, BH), lambda i: (0, i)),
        out_shape=jax.ShapeDtypeStruct((B, H), jnp.float32),
    )(batch, W1, b1.reshape(1, H))

    BD = 1024
    op = pl.pallas_call(
        _head_kernel,
        grid=(D // BD,),
        in_specs=[
            pl.BlockSpec((B, H), lambda i: (0, 0)),
            pl.BlockSpec((BD, H), lambda i: (i, 0)),
            pl.BlockSpec((BD, H), lambda i: (i, 0)),
            pl.BlockSpec((1, BD), lambda i: (0, i)),
            pl.BlockSpec((1, BD), lambda i: (0, i)),
            pl.BlockSpec((B, BD), lambda i: (0, i)),
            pl.BlockSpec((B, BD), lambda i: (0, i)),
            pl.BlockSpec((1, 1), lambda i: (0, 0), memory_space=pltpu.SMEM),
        ],
        out_specs=pl.BlockSpec((B, BD), lambda i: (0, i)),
        out_shape=jax.ShapeDtypeStruct((B, D), jnp.float32),
    )(h, W21, W22, b21.reshape(1, D), b22.reshape(1, D), e_sum, batch, scale)

    nsc = B // 2
    out_sc = _make_sc_topk(nsc, D, k, nsc // 32)(op[:nsc])
    out_tc = pl.pallas_call(
        _make_topk_kernel(k),
        out_shape=jax.ShapeDtypeStruct((B - nsc, D), jnp.float32),
    )(op[nsc:])
    return jnp.concatenate([out_sc, out_tc], axis=0)
